# baseline (speedup 1.0000x reference)
import functools

import jax
import jax.numpy as jnp
from jax import lax
from jax.experimental import pallas as pl
from jax.experimental.pallas import tpu as pltpu
from jax.experimental.pallas import tpu_sc as plsc

_C = 128
_NBUF = 4
_LA = 2


@functools.lru_cache(maxsize=None)
def _make_sc_gather(B, V, D, dtype_name):
    dtype = jnp.dtype(dtype_name)
    info = plsc.get_sparse_core_info()
    NC, NS = info.num_cores, info.num_subcores
    NW = NC * NS
    assert B % NW == 0
    b_per_w = B // NW
    C, NBUF = _C, _NBUF
    assert b_per_w % C == 0
    n_chunks = b_per_w // C
    assert n_chunks % NBUF == 0 and n_chunks >= 2 * NBUF
    mesh = plsc.VectorSubcoreMesh(core_axis_name="c", subcore_axis_name="s")

    @functools.partial(
        pl.kernel,
        mesh=mesh,
        out_type=jax.ShapeDtypeStruct((B, D), dtype),
        scratch_types=[
            pltpu.VMEM((b_per_w,), jnp.int32),
            pltpu.VMEM((NBUF, C, D), dtype),
            pltpu.SemaphoreType.DMA((NBUF,)),
            pltpu.SemaphoreType.DMA((NBUF,)),
        ],
    )
    def k(table_hbm, idx_hbm, out_hbm, idx_v, rows, gsem, ssem):
        wid = lax.axis_index("s") * NC + lax.axis_index("c")
        base = wid * b_per_w
        pltpu.sync_copy(idx_hbm.at[pl.ds(base, b_per_w)], idx_v)

        def start_gather(g, b):
            pltpu.async_copy(
                table_hbm.at[idx_v.at[pl.ds(g * C, C)]], rows.at[b], gsem.at[b]
            )

        def wait_gather(b):
            pltpu.make_async_copy(
                table_hbm.at[idx_v.at[pl.ds(0, C)]], rows.at[b], gsem.at[b]
            ).wait()

        def start_store(g, b):
            pltpu.async_copy(
                rows.at[b], out_hbm.at[pl.ds(base + g * C, C)], ssem.at[b]
            )

        def wait_store(b):
            pltpu.make_async_copy(
                rows.at[b], out_hbm.at[pl.ds(base, C)], ssem.at[b]
            ).wait()

        for g in range(NBUF):
            start_gather(g, g)
            if g >= _LA:
                wait_gather(g - _LA)
                start_store(g - _LA, g - _LA)

        def body(o, carry):
            g0 = o * NBUF
            for b in range(NBUF):
                g = g0 + b
                wait_store(b)
                start_gather(g, b)
                wait_gather((b - _LA) % NBUF)
                start_store(g - _LA, (b - _LA) % NBUF)
            return carry

        lax.fori_loop(1, n_chunks // NBUF, body, 0)

        for g in range(n_chunks - _LA, n_chunks):
            wait_gather(g % NBUF)
            start_store(g, g % NBUF)
        for b in range(NBUF):
            wait_store(b)

    return k


def kernel(ids, weight_fp):
    V, D = weight_fp.shape
    ids_flat = ids.reshape(-1).astype(jnp.int32)
    B = ids_flat.shape[0]
    B_sc = (B * 9 // 10) // (32 * _C * _NBUF) * (32 * _C * _NBUF)
    out_sc = _make_sc_gather(B_sc, V, D, weight_fp.dtype.name)(
        weight_fp, ids_flat[:B_sc]
    )
    out_tc = jnp.take(weight_fp, ids_flat[B_sc:], axis=0)
    out = jnp.concatenate([out_sc, out_tc], axis=0)
    return out.reshape(*ids.shape, D)


# speedup vs baseline: 1.9089x; 1.9089x over previous
import functools

import jax
import jax.numpy as jnp
from jax import lax
from jax.experimental import pallas as pl
from jax.experimental.pallas import tpu as pltpu
from jax.experimental.pallas import tpu_sc as plsc

_C = 80
_NBUF = 8
_LA = 4


@functools.lru_cache(maxsize=None)
def _make_sc_gather(B, V, D, dtype_name):
    dtype = jnp.dtype(dtype_name)
    info = plsc.get_sparse_core_info()
    NC, NS = info.num_cores, info.num_subcores
    NW = NC * NS
    assert B % NW == 0
    b_per_w = B // NW
    C, NBUF = _C, _NBUF
    assert b_per_w % C == 0
    n_chunks = b_per_w // C
    assert n_chunks % NBUF == 0 and n_chunks >= 2 * NBUF
    mesh = plsc.VectorSubcoreMesh(core_axis_name="c", subcore_axis_name="s")

    @functools.partial(
        pl.kernel,
        mesh=mesh,
        out_type=jax.ShapeDtypeStruct((B, D), dtype),
        scratch_types=[
            pltpu.VMEM((b_per_w,), jnp.int32),
            pltpu.VMEM((NBUF, C, D), dtype),
            pltpu.SemaphoreType.DMA((NBUF,)),
            pltpu.SemaphoreType.DMA((NBUF,)),
        ],
    )
    def k(table_hbm, idx_hbm, out_hbm, idx_v, rows, gsem, ssem):
        wid = lax.axis_index("s") * NC + lax.axis_index("c")
        base = wid * b_per_w
        pltpu.sync_copy(idx_hbm.at[pl.ds(base, b_per_w)], idx_v)

        def start_gather(g, b):
            pltpu.async_copy(
                table_hbm.at[idx_v.at[pl.ds(g * C, C)]], rows.at[b], gsem.at[b]
            )

        def wait_gather(b):
            pltpu.make_async_copy(
                table_hbm.at[idx_v.at[pl.ds(0, C)]], rows.at[b], gsem.at[b]
            ).wait()

        def start_store(g, b):
            pltpu.async_copy(
                rows.at[b], out_hbm.at[pl.ds(base + g * C, C)], ssem.at[b]
            )

        def wait_store(b):
            pltpu.make_async_copy(
                rows.at[b], out_hbm.at[pl.ds(base, C)], ssem.at[b]
            ).wait()

        for g in range(NBUF):
            start_gather(g, g)
            if g >= _LA:
                wait_gather(g - _LA)
                start_store(g - _LA, g - _LA)

        def body(o, carry):
            g0 = o * NBUF
            for b in range(NBUF):
                g = g0 + b
                wait_store(b)
                start_gather(g, b)
                wait_gather((b - _LA) % NBUF)
                start_store(g - _LA, (b - _LA) % NBUF)
            return carry

        lax.fori_loop(1, n_chunks // NBUF, body, 0)

        for g in range(n_chunks - _LA, n_chunks):
            wait_gather(g % NBUF)
            start_store(g, g % NBUF)
        for b in range(NBUF):
            wait_store(b)

    return k


def kernel(ids, weight_fp):
    V, D = weight_fp.shape
    ids_flat = ids.reshape(-1).astype(jnp.int32)
    B = ids_flat.shape[0]
    out = _make_sc_gather(B, V, D, weight_fp.dtype.name)(weight_fp, ids_flat)
    return out.reshape(*ids.shape, D)


# speedup vs baseline: 1.9877x; 1.0413x over previous
import functools

import jax
import jax.numpy as jnp
from jax import lax
from jax.experimental import pallas as pl
from jax.experimental.pallas import tpu as pltpu
from jax.experimental.pallas import tpu_sc as plsc

_C = 128
_NB = 4
_NP = 2
_LA = 2


@functools.lru_cache(maxsize=None)
def _make_sc_gather(B, V, D, dtype_name):
    dtype = jnp.dtype(dtype_name)
    info = plsc.get_sparse_core_info()
    NC, NS = info.num_cores, info.num_subcores
    NW = NC * NS
    assert B % NW == 0
    b_per_w = B // NW
    C, NB, NP = _C, _NB, _NP
    assert b_per_w % C == 0
    n_chunks = b_per_w // C
    assert n_chunks % NB == 0 and n_chunks >= 2 * NB
    mesh = plsc.VectorSubcoreMesh(core_axis_name="c", subcore_axis_name="s")

    @functools.partial(
        pl.kernel,
        mesh=mesh,
        out_type=jax.ShapeDtypeStruct((B, D), dtype),
        scratch_types=[
            pltpu.VMEM((b_per_w,), jnp.int32),
            pltpu.VMEM((NB, C, D), dtype),
            pltpu.VMEM_SHARED((NS, NP, C, D), dtype),
            pltpu.SemaphoreType.DMA((NB,)),
            pltpu.SemaphoreType.DMA((NP,)),
            pltpu.SemaphoreType.DMA((NP,)),
        ],
    )
    def k(table_hbm, idx_hbm, out_hbm, idx_v, rows, shared, gsem, csem, ssem):
        cid = lax.axis_index("c")
        sid = lax.axis_index("s")
        wid = sid * NC + cid
        base = wid * b_per_w
        pltpu.sync_copy(idx_hbm.at[pl.ds(base, b_per_w)], idx_v)

        def start_gather(g, b):
            pltpu.async_copy(
                table_hbm.at[idx_v.at[pl.ds(g * C, C)]], rows.at[b], gsem.at[b]
            )

        def wait_gather(b):
            pltpu.make_async_copy(
                table_hbm.at[idx_v.at[pl.ds(0, C)]], rows.at[b], gsem.at[b]
            ).wait()

        def start_cross(b, p):
            pltpu.async_copy(rows.at[b], shared.at[sid, p], csem.at[p])

        def wait_cross(p):
            pltpu.make_async_copy(
                rows.at[0], shared.at[sid, p], csem.at[p]
            ).wait()

        def start_store(g, p):
            pltpu.async_copy(
                shared.at[sid, p], out_hbm.at[pl.ds(base + g * C, C)], ssem.at[p]
            )

        def wait_store(p):
            pltpu.make_async_copy(
                shared.at[sid, p], out_hbm.at[pl.ds(base, C)], ssem.at[p]
            ).wait()

        start_gather(0, 0)
        start_gather(1, 1)
        for i in range(NB):
            b, p = i % NB, i % NP
            wait_gather(b)
            if i >= NP:
                wait_store(p)
            start_cross(b, p)
            if i >= 1:
                wait_cross((p - 1) % NP)
                start_store(i - 1, (p - 1) % NP)
            if i + _LA < n_chunks:
                start_gather(i + _LA, (i + _LA) % NB)

        def body(o, carry):
            i0 = o * NB
            for b in range(NB):
                i = i0 + b
                p = b % NP
                wait_gather(b)
                wait_store(p)
                start_cross(b, p)
                wait_cross((p - 1) % NP)
                start_store(i - 1, (p - 1) % NP)
                start_gather(i + _LA, (b + _LA) % NB)
            return carry

        lax.fori_loop(1, n_chunks // NB - 1, body, 0)

        for i in range(n_chunks - NB, n_chunks):
            b, p = i % NB, i % NP
            wait_gather(b)
            wait_store(p)
            start_cross(b, p)
            wait_cross((p - 1) % NP)
            start_store(i - 1, (p - 1) % NP)
            if i + _LA < n_chunks:
                start_gather(i + _LA, (b + _LA) % NB)
        wait_cross((n_chunks - 1) % NP)
        start_store(n_chunks - 1, (n_chunks - 1) % NP)
        for p in range(NP):
            wait_store(p)

    return k


def kernel(ids, weight_fp):
    V, D = weight_fp.shape
    ids_flat = ids.reshape(-1).astype(jnp.int32)
    B = ids_flat.shape[0]
    out = _make_sc_gather(B, V, D, weight_fp.dtype.name)(weight_fp, ids_flat)
    return out.reshape(*ids.shape, D)
